# baseline (speedup 1.0000x reference)
import functools
import math

import jax
import jax.numpy as jnp
from jax import lax
from jax.experimental import pallas as pl
from jax.experimental.pallas import tpu as pltpu
from jax.experimental.pallas import tpu_sc as plsc

_EMBED_DIM = 64
_SCALE = math.sqrt(_EMBED_DIM)

_NC = 2
_NS = 16
_NW = _NC * _NS

_CI = 2
_NBUF = 2


def _make_kernel(V_pad, R, S):
  r_per_w = R // _NW
  chunks = r_per_w // _CI
  halves = chunks // _NBUF
  chunk_idx = _CI * S
  rows_per_tile = V_pad // _NS

  mesh = plsc.VectorSubcoreMesh(core_axis_name="c", subcore_axis_name="s",
                                num_cores=_NC, num_subcores=_NS)

  @functools.partial(
      pl.kernel,
      mesh=mesh,
      compiler_params=pltpu.CompilerParams(use_tc_tiling_on_sc=False),
      out_type=jax.ShapeDtypeStruct((R, S, _EMBED_DIM), jnp.float32),
      scratch_types=[
          pltpu.VMEM_SHARED((V_pad, _EMBED_DIM), jnp.float32),
          pltpu.VMEM((rows_per_tile, _EMBED_DIM), jnp.float32),
          pltpu.VMEM((_NBUF, chunk_idx), jnp.int32),
          pltpu.VMEM((_NBUF, chunk_idx, _EMBED_DIM), jnp.float32),
          [pltpu.SemaphoreType.DMA] * _NBUF,
          [pltpu.SemaphoreType.DMA] * _NBUF,
          [pltpu.SemaphoreType.DMA] * _NBUF,
      ],
  )
  def k(lut_hbm, idx_hbm, out_hbm, table_sh, scale_v, idx_v, rows_v,
        sems_i, sems_g, sems_w):
    cid = lax.axis_index("c")
    sid = lax.axis_index("s")
    wid = sid * _NC + cid

    row0 = sid * rows_per_tile
    pltpu.sync_copy(lut_hbm.at[pl.ds(row0, rows_per_tile)], scale_v)

    def scale_row(i, _):
      for j in range(_EMBED_DIM // 16):
        scale_v[i, pl.ds(j * 16, 16)] = scale_v[i, pl.ds(j * 16, 16)] * _SCALE
      return 0

    lax.fori_loop(0, rows_per_tile, scale_row, 0)
    pltpu.sync_copy(scale_v, table_sh.at[pl.ds(row0, rows_per_tile)])
    plsc.subcore_barrier()

    flat_base = wid * r_per_w * S
    out_base = wid * r_per_w

    def idx_copy(g, b):
      return pltpu.make_async_copy(
          idx_hbm.at[pl.ds(flat_base + g * chunk_idx, chunk_idx)],
          idx_v.at[b], sems_i[b])

    def gather_copy(b, a):
      return pltpu.make_async_copy(
          lut_hbm.at[idx_v.at[b].at[pl.ds(a * S, S)]],
          rows_v.at[b].at[pl.ds(a * S, S)], sems_g[b])

    def out_copy(g, b, a):
      return pltpu.make_async_copy(
          rows_v.at[b].at[pl.ds(a * S, S)],
          out_hbm.at[out_base + g * _CI + a], sems_w[b])

    for b in range(_NBUF):
      idx_copy(b, b).start()

    def body(h, _):
      for b in range(_NBUF):
        g = h * _NBUF + b
        bp = (b - 1) % _NBUF
        idx_copy(g, b).wait()

        @pl.when(h > 0)
        def _():
          for a in range(_CI):
            out_copy(g, b, a).wait()

        for a in range(_CI):
          gather_copy(b, a).start()

        @pl.when(g > 0)
        def _():
          for a in range(_CI):
            gather_copy(bp, a).wait()
          for a in range(_CI):
            out_copy(g - 1, bp, a).start()
          @pl.when(g - 1 + _NBUF < chunks)
          def _():
            idx_copy(g - 1 + _NBUF, bp).start()
      return 0

    lax.fori_loop(0, halves, body, 0)

    bl = (chunks - 1) % _NBUF
    for a in range(_CI):
      gather_copy(bl, a).wait()
    for a in range(_CI):
      out_copy(chunks - 1, bl, a).start()
    for b in range(_NBUF):
      for a in range(_CI):
        out_copy(chunks - _NBUF + b, b, a).wait()

  return k


def kernel(x, lut):
  V, D = lut.shape
  R, S = x.shape
  V_pad = -(-V // (_NS * 8)) * (_NS * 8)
  lut_pad = jnp.pad(lut * _SCALE, ((0, V_pad - V), (0, 0)))
  idx_flat = x.reshape(-1).astype(jnp.int32)
  return _make_kernel(V_pad, R, S)(lut_pad, idx_flat)


# speedup vs baseline: 1.2993x; 1.2993x over previous
import functools
import math

import jax
import jax.numpy as jnp
from jax import lax
from jax.experimental import pallas as pl
from jax.experimental.pallas import tpu as pltpu
from jax.experimental.pallas import tpu_sc as plsc

_EMBED_DIM = 64
_WIDE = 128
_SCALE = math.sqrt(_EMBED_DIM)

_NC = 2
_NS = 16
_NW = _NC * _NS

_H_OFF = (0, 96)
_H_CNT = (96, 104)
_BUF = 104


def _make_kernel(V_pad, R, S):
  r_per_w = R // _NW
  row_pairs = r_per_w // 2
  stage_cnt = (96, 96, 104)
  stage_off = (0, 96, 192)
  assert sum(stage_cnt) == V_pad // _NS

  mesh = plsc.VectorSubcoreMesh(core_axis_name="c", subcore_axis_name="s",
                                num_cores=_NC, num_subcores=_NS)

  @functools.partial(
      pl.kernel,
      mesh=mesh,
      out_type=jax.ShapeDtypeStruct((R, S, _EMBED_DIM), jnp.float32),
      scratch_types=[
          pltpu.VMEM_SHARED((V_pad, _WIDE), jnp.float32),
          [pltpu.VMEM((_BUF,), jnp.int32)] * 4,
          [pltpu.VMEM((_BUF, _WIDE), jnp.float32)] * 2,
          [pltpu.VMEM((_BUF, _EMBED_DIM), jnp.float32)] * 2,
          [pltpu.SemaphoreType.DMA] * 4,
          [pltpu.SemaphoreType.DMA] * 2,
          [pltpu.SemaphoreType.DMA] * 2,
      ],
  )
  def k(lut_hbm, idx_hbm, out_hbm, table_sh, idxs, wides, packs,
        sems_i, sems_g, sems_w):
    cid = lax.axis_index("c")
    sid = lax.axis_index("s")
    wid = sid * _NC + cid

    tile_row0 = sid * (V_pad // _NS)
    for t in range(3):
      cnt = stage_cnt[t]
      row0 = tile_row0 + stage_off[t]
      stage = wides[0].at[pl.ds(0, cnt)]
      pltpu.sync_copy(lut_hbm.at[pl.ds(row0, cnt)], stage)

      def scale_row(i, _):
        for j in range(_WIDE // 16):
          wides[0][i, pl.ds(j * 16, 16)] = (
              wides[0][i, pl.ds(j * 16, 16)] * _SCALE)
        return 0

      lax.fori_loop(0, cnt, scale_row, 0)
      pltpu.sync_copy(stage, table_sh.at[pl.ds(row0, cnt)])
    plsc.subcore_barrier()

    row_base = wid * r_per_w
    idx_base = wid * r_per_w * S

    def idx_copy(r, h, b):
      return pltpu.make_async_copy(
          idx_hbm.at[pl.ds(idx_base + r * S + _H_OFF[h], _H_CNT[h])],
          idxs[b].at[pl.ds(0, _H_CNT[h])], sems_i[b])

    def gather_copy(b, h):
      return pltpu.make_async_copy(
          table_sh.at[idxs[b].at[pl.ds(0, _H_CNT[h])]],
          wides[h].at[pl.ds(0, _H_CNT[h])], sems_g[h])

    def repack(h):
      def rp_row(i, _):
        for j in range(_EMBED_DIM // 16):
          packs[h][i, pl.ds(j * 16, 16)] = wides[h][i, pl.ds(j * 16, 16)]
        return 0
      lax.fori_loop(0, _H_CNT[h], rp_row, 0)

    def out_copy(r, h):
      return pltpu.make_async_copy(
          packs[h].at[pl.ds(0, _H_CNT[h])],
          out_hbm.at[row_base + r, pl.ds(_H_OFF[h], _H_CNT[h])],
          sems_w[h])

    for b in range(3):
      idx_copy(b // 2, b % 2, b).start()

    pf = ((1, 1, 3), (2, 0, 0), (2, 1, 1), (3, 0, 2))

    def body(g, _):
      for rr in range(2):
        r = g * 2 + rr
        for h in range(2):
          b = 2 * rr + h
          hp = 1 - h
          idx_copy(r, h, b).wait()

          if b >= 2:
            out_copy(r, h).wait()
          else:
            @pl.when(g > 0)
            def _():
              out_copy(r, h).wait()

          gather_copy(b, h).start()

          rp = r if h == 1 else r - 1
          if b >= 1:
            gather_copy((b + 3) % 4, hp).wait()
            repack(hp)
            out_copy(rp, hp).start()
          else:
            @pl.when(g > 0)
            def _():
              gather_copy((b + 3) % 4, hp).wait()
              repack(hp)
              out_copy(rp, hp).start()

          dr, nh, slot = pf[b]
          if b == 0:
            idx_copy(g * 2 + dr, nh, slot).start()
          else:
            @pl.when(g + 1 < row_pairs)
            def _():
              idx_copy(g * 2 + dr, nh, slot).start()
      return 0

    lax.fori_loop(0, row_pairs, body, 0)

    gather_copy(3, 1).wait()
    repack(1)
    out_copy(r_per_w - 1, 1).start()
    out_copy(r_per_w - 1, 0).wait()
    out_copy(r_per_w - 1, 1).wait()

  return k


def kernel(x, lut):
  V, D = lut.shape
  R, S = x.shape
  V_pad = -(-V // (_NS * 8)) * (_NS * 8)
  lut_pad = jnp.pad(lut, ((0, V_pad - V), (0, _WIDE - D)))
  idx_flat = x.reshape(-1).astype(jnp.int32)
  return _make_kernel(V_pad, R, S)(lut_pad, idx_flat)
